# baseline (speedup 1.0000x reference)
import functools

import jax
import jax.numpy as jnp
from jax import lax
from jax.experimental import pallas as pl
from jax.experimental.pallas import tpu as pltpu
from jax.experimental.pallas import tpu_sc as plsc

N = 10000
E = 320000
D = 128
NCORE = 2
NSUB = 16
NW = NCORE * NSUB
CHUNK = 128
CPW = 80
BLK = 8
NBLK = CPW // BLK
NCHUNK = NW * CPW
EPAD = NCHUNK * CHUNK
NTRASH = 64
NPAD = N + NTRASH
RPS = 624
TAIL = N - NSUB * RPS
RB = 1000
GRID = N // RB
EPS = 1e-5

def _deg_build():
    @functools.partial(
        pl.kernel,
        mesh=plsc.VectorSubcoreMesh(core_axis_name="c", subcore_axis_name="s"),
        out_type=jax.ShapeDtypeStruct((NCORE, N, D), jnp.float32),
        scratch_types=[
            pltpu.VMEM_SHARED((NPAD, D), jnp.float32),
            pltpu.VMEM((CPW, CHUNK), jnp.int32),
            pltpu.VMEM((CHUNK, D), jnp.float32),
            pltpu.SemaphoreType.DMA,
            pltpu.SemaphoreType.DMA,
        ],
    )
    def degk(dst_hbm, z_hbm, ones_hbm, out_hbm, acc_sp, didx, ones_v,
             sem_s0, sem_s1):
        cid = lax.axis_index("c")
        sid = lax.axis_index("s")
        wid = cid * NSUB + sid
        base = sid * RPS
        pltpu.sync_copy(z_hbm.at[pl.ds(base, RPS)], acc_sp.at[pl.ds(base, RPS)])
        pltpu.sync_copy(ones_hbm, ones_v)
        pltpu.sync_copy(dst_hbm.at[pl.ds(wid * CPW, CPW)], didx)

        @pl.when(sid == 0)
        def _():
            t0 = NSUB * RPS
            pltpu.sync_copy(z_hbm.at[pl.ds(t0, TAIL)], acc_sp.at[pl.ds(t0, TAIL)])
        plsc.subcore_barrier()

        def step(jj, carry):
            j0 = jj * 2
            j1 = j0 + 1

            @pl.when(jj > 0)
            def _():
                pltpu.make_async_copy(
                    ones_v, acc_sp.at[didx.at[j0 - 2]], sem_s0).wait()
                pltpu.make_async_copy(
                    ones_v, acc_sp.at[didx.at[j1 - 2]], sem_s1).wait()
            pltpu.async_copy(ones_v, acc_sp.at[didx.at[j0]], sem_s0, add=True)
            pltpu.async_copy(ones_v, acc_sp.at[didx.at[j1]], sem_s1, add=True)
            return carry

        lax.fori_loop(0, CPW // 2, step, 0)
        pltpu.make_async_copy(ones_v, acc_sp.at[didx.at[CPW - 2]], sem_s0).wait()
        pltpu.make_async_copy(ones_v, acc_sp.at[didx.at[CPW - 1]], sem_s1).wait()
        plsc.subcore_barrier()
        pltpu.sync_copy(acc_sp.at[pl.ds(base, RPS)],
                        out_hbm.at[cid, pl.ds(base, RPS)])

        @pl.when(sid == 0)
        def _():
            t0 = NSUB * RPS
            pltpu.sync_copy(acc_sp.at[pl.ds(t0, TAIL)],
                            out_hbm.at[cid, pl.ds(t0, TAIL)])

    return degk


def _agg_build():
    @functools.partial(
        pl.kernel,
        mesh=plsc.VectorSubcoreMesh(core_axis_name="c", subcore_axis_name="s"),
        out_type=jax.ShapeDtypeStruct((NCORE, N, D), jnp.float32),
        scratch_types=[
            pltpu.VMEM_SHARED((NPAD, D), jnp.float32),
            pltpu.VMEM((2, BLK, CHUNK), jnp.int32),
            pltpu.VMEM((2, BLK, CHUNK), jnp.int32),
            pltpu.VMEM((CHUNK, D), jnp.float32),
            pltpu.VMEM((CHUNK, D), jnp.float32),
            pltpu.SemaphoreType.DMA,
            pltpu.SemaphoreType.DMA,
            pltpu.SemaphoreType.DMA,
            pltpu.SemaphoreType.DMA,
        ],
    )
    def agg(h_hbm, src_hbm, dst_hbm, z_hbm,
            out_hbm,
            acc_sp, sidx, didx, rows0, rows1,
            sem_g0, sem_g1, sem_s0, sem_s1):
        cid = lax.axis_index("c")
        sid = lax.axis_index("s")
        wid = cid * NSUB + sid
        base = sid * RPS
        pltpu.sync_copy(z_hbm.at[pl.ds(base, RPS)], acc_sp.at[pl.ds(base, RPS)])
        pltpu.sync_copy(src_hbm.at[pl.ds(wid * CPW, BLK)], sidx.at[0])
        pltpu.sync_copy(dst_hbm.at[pl.ds(wid * CPW, BLK)], didx.at[0])

        @pl.when(sid == 0)
        def _():
            t0 = NSUB * RPS
            pltpu.sync_copy(z_hbm.at[pl.ds(t0, TAIL)], acc_sp.at[pl.ds(t0, TAIL)])
        plsc.subcore_barrier()

        rows = (rows0, rows1)
        sem_g = (sem_g0, sem_g1)
        sem_s = (sem_s0, sem_s1)
        pltpu.async_copy(h_hbm.at[sidx.at[0, 0]], rows0, sem_g0)

        def outer(ob, carry):
            for bb in (0, 1):
                blk = ob * 2 + bb
                for k in range(BLK):
                    p = k % 2
                    if k == 0:
                        @pl.when(blk > 0)
                        def _():
                            pltpu.make_async_copy(
                                rows[1 - p],
                                acc_sp.at[didx.at[1 - bb, BLK - 1]],
                                sem_s[1 - p]).wait()
                        @pl.when(blk < NBLK - 1)
                        def _():
                            off = wid * CPW + (blk + 1) * BLK
                            pltpu.sync_copy(src_hbm.at[pl.ds(off, BLK)],
                                            sidx.at[1 - bb])
                            pltpu.sync_copy(dst_hbm.at[pl.ds(off, BLK)],
                                            didx.at[1 - bb])
                    else:
                        pltpu.make_async_copy(
                            rows[1 - p], acc_sp.at[didx.at[bb, k - 1]],
                            sem_s[1 - p]).wait()
                    if k == BLK - 1:
                        @pl.when(blk < NBLK - 1)
                        def _():
                            pltpu.async_copy(h_hbm.at[sidx.at[1 - bb, 0]],
                                             rows[1 - p], sem_g[1 - p])
                    else:
                        pltpu.async_copy(h_hbm.at[sidx.at[bb, k + 1]],
                                         rows[1 - p], sem_g[1 - p])
                    pltpu.make_async_copy(h_hbm.at[sidx.at[bb, k]],
                                          rows[p], sem_g[p]).wait()
                    pltpu.async_copy(rows[p], acc_sp.at[didx.at[bb, k]],
                                     sem_s[p], add=True)
            return carry

        lax.fori_loop(0, NBLK // 2, outer, 0)
        pltpu.make_async_copy(rows1, acc_sp.at[didx.at[1, BLK - 1]],
                              sem_s1).wait()
        plsc.subcore_barrier()
        pltpu.sync_copy(acc_sp.at[pl.ds(base, RPS)],
                        out_hbm.at[cid, pl.ds(base, RPS)])

        @pl.when(sid == 0)
        def _():
            t0 = NSUB * RPS
            pltpu.sync_copy(acc_sp.at[pl.ds(t0, TAIL)],
                            out_hbm.at[cid, pl.ds(t0, TAIL)])

    return agg


_sc_cache = {}


def _deg(*args):
    if "deg" not in _sc_cache:
        _sc_cache["deg"] = _deg_build()
    return _sc_cache["deg"](*args)


def _agg(*args):
    if "agg" not in _sc_cache:
        _sc_cache["agg"] = _agg_build()
    return _sc_cache["agg"](*args)


def _deginv_body(dacc_ref, out_ref):
    d = dacc_ref[0, :, 0:1] + dacc_ref[1, :, 0:1]
    out_ref[...] = jnp.broadcast_to(1.0 / jnp.clip(d, 1.0, None), (RB, D))


def _deginv(dacc):
    return pl.pallas_call(
        _deginv_body,
        grid=(GRID,),
        in_specs=[pl.BlockSpec((NCORE, RB, D), lambda i: (0, i, 0))],
        out_specs=pl.BlockSpec((RB, D), lambda i: (i, 0)),
        out_shape=jax.ShapeDtypeStruct((N, D), jnp.float32),
    )(dacc)


def _hr_body(h_ref, wr_ref, b_ref, out_ref):
    out_ref[...] = jnp.dot(h_ref[...], wr_ref[...],
                           preferred_element_type=jnp.float32) + b_ref[...]


def _hr(h, wr, b):
    return pl.pallas_call(
        _hr_body,
        grid=(GRID,),
        in_specs=[
            pl.BlockSpec((RB, D), lambda i: (i, 0)),
            pl.BlockSpec((D, D), lambda i: (0, 0)),
            pl.BlockSpec((1, D), lambda i: (0, 0)),
        ],
        out_specs=pl.BlockSpec((RB, D), lambda i: (i, 0)),
        out_shape=jax.ShapeDtypeStruct((N, D), jnp.float32),
    )(h, wr, b)


def _layer_body(acc_ref, dinv_ref, hr_ref, wl_ref, g_ref,
                beta_ref, out_ref, pre_scr, s1_scr, s2_scr):
    ph = pl.program_id(0)
    i = pl.program_id(1)

    @pl.when(ph == 0)
    def _():
        mean = (acc_ref[0] + acc_ref[1]) * dinv_ref[...]
        pre = jnp.dot(mean, wl_ref[...], preferred_element_type=jnp.float32)
        pre = pre + hr_ref[...]
        pre_scr[pl.ds(i * RB, RB), :] = pre

        @pl.when(i == 0)
        def _():
            s1_scr[...] = jnp.zeros((1, D), jnp.float32)
            s2_scr[...] = jnp.zeros((1, D), jnp.float32)
        s1_scr[...] += jnp.sum(pre, axis=0, keepdims=True)
        s2_scr[...] += jnp.sum(pre * pre, axis=0, keepdims=True)

    @pl.when(ph == 1)
    def _():
        mu = s1_scr[...] / N
        var = s2_scr[...] / N - mu * mu
        scale = g_ref[...] * lax.rsqrt(var + EPS)
        pre = pre_scr[pl.ds(i * RB, RB), :]
        out_ref[...] = jnp.maximum((pre - mu) * scale + beta_ref[...], 0.0)


def _layer(acc, dinv, hr, wl, g, beta):
    blk = lambda ph, i: (i * (1 - ph), 0)
    return pl.pallas_call(
        _layer_body,
        grid=(2, GRID),
        in_specs=[
            pl.BlockSpec((NCORE, RB, D), lambda ph, i: (0, i * (1 - ph), 0)),
            pl.BlockSpec((RB, D), blk),
            pl.BlockSpec((RB, D), blk),
            pl.BlockSpec((D, D), lambda ph, i: (0, 0)),
            pl.BlockSpec((1, D), lambda ph, i: (0, 0)),
            pl.BlockSpec((1, D), lambda ph, i: (0, 0)),
        ],
        out_specs=pl.BlockSpec((RB, D), lambda ph, i: (i, 0)),
        out_shape=jax.ShapeDtypeStruct((N, D), jnp.float32),
        scratch_shapes=[
            pltpu.VMEM((N, D), jnp.float32),
            pltpu.VMEM((1, D), jnp.float32),
            pltpu.VMEM((1, D), jnp.float32),
        ],
    )(acc, dinv, hr, wl, g, beta)


def _out_body(acc_ref, dinv_ref, hr_ref, wl_ref, out_ref):
    mean = (acc_ref[0] + acc_ref[1]) * dinv_ref[...]
    pre = jnp.dot(mean, wl_ref[...], preferred_element_type=jnp.float32)
    pre = pre + hr_ref[...]
    m = jnp.max(pre, axis=1, keepdims=True)
    e = jnp.exp(pre - m)
    s = jnp.sum(e, axis=1, keepdims=True)
    out_ref[...] = pre - m - jnp.log(s)


def _out_layer(acc, dinv, hr, wl):
    return pl.pallas_call(
        _out_body,
        grid=(GRID,),
        in_specs=[
            pl.BlockSpec((NCORE, RB, D), lambda i: (0, i, 0)),
            pl.BlockSpec((RB, D), lambda i: (i, 0)),
            pl.BlockSpec((RB, D), lambda i: (i, 0)),
            pl.BlockSpec((D, D), lambda i: (0, 0)),
        ],
        out_specs=pl.BlockSpec((RB, D), lambda i: (i, 0)),
        out_shape=jax.ShapeDtypeStruct((N, D), jnp.float32),
    )(acc, dinv, hr, wl)


def kernel(x, edge_index, Wl1, Wr1, b1, g1, beta1, Wl2, Wr2, b2, g2, beta2,
           Wl3, Wr3, b3, g3, beta3, Wl4, Wr4, b4):
    pad = EPAD - E
    tr = jnp.arange(pad, dtype=jnp.int32) % NTRASH
    src = jnp.concatenate([edge_index[0], tr]).reshape(NCHUNK, CHUNK)
    dst = jnp.concatenate([edge_index[1], N + tr]).reshape(NCHUNK, CHUNK)
    zeros = jnp.zeros((N, D), jnp.float32)
    ones = jnp.ones((CHUNK, D), jnp.float32)
    r = lambda v: v.reshape(1, D)

    dacc = _deg(dst, zeros, ones)
    acc1 = _agg(x, src, dst, zeros)
    hr1 = _hr(x, Wr1, r(b1))
    dinv = _deginv(dacc)
    h1 = _layer(acc1, dinv, hr1, Wl1, r(g1), r(beta1))

    acc2 = _agg(h1, src, dst, zeros)
    hr2 = _hr(h1, Wr2, r(b2))
    h2 = _layer(acc2, dinv, hr2, Wl2, r(g2), r(beta2))

    acc3 = _agg(h2, src, dst, zeros)
    hr3 = _hr(h2, Wr3, r(b3))
    h3 = _layer(acc3, dinv, hr3, Wl3, r(g3), r(beta3))

    acc4 = _agg(h3, src, dst, zeros)
    hr4 = _hr(h3, Wr4, r(b4))
    return _out_layer(acc4, dinv, hr4, Wl4)


# speedup vs baseline: 1.0254x; 1.0254x over previous
import functools

import jax
import jax.numpy as jnp
from jax import lax
from jax.experimental import pallas as pl
from jax.experimental.pallas import tpu as pltpu
from jax.experimental.pallas import tpu_sc as plsc

N = 10000
E = 320000
D = 128
NCORE = 2
NSUB = 16
NW = NCORE * NSUB
CHUNK = 128
CPW = 80
BLK = 16
NBLK = CPW // BLK
NCHUNK = NW * CPW
EPAD = NCHUNK * CHUNK
NTRASH = 64
NPAD = N + NTRASH
RPS = 624
TAIL = N - NSUB * RPS
RB = 1000
GRID = N // RB
EPS = 1e-5

def _deg_build():
    @functools.partial(
        pl.kernel,
        mesh=plsc.VectorSubcoreMesh(core_axis_name="c", subcore_axis_name="s"),
        out_type=jax.ShapeDtypeStruct((NCORE, N, D), jnp.float32),
        scratch_types=[
            pltpu.VMEM_SHARED((NPAD, D), jnp.float32),
            pltpu.VMEM((CPW, CHUNK), jnp.int32),
            pltpu.VMEM((CHUNK, D), jnp.float32),
            pltpu.SemaphoreType.DMA,
            pltpu.SemaphoreType.DMA,
        ],
    )
    def degk(dst_hbm, z_hbm, ones_hbm, out_hbm, acc_sp, didx, ones_v,
             sem_s0, sem_s1):
        cid = lax.axis_index("c")
        sid = lax.axis_index("s")
        wid = cid * NSUB + sid
        base = sid * RPS
        pltpu.sync_copy(z_hbm.at[pl.ds(base, RPS)], acc_sp.at[pl.ds(base, RPS)])
        pltpu.sync_copy(ones_hbm, ones_v)
        pltpu.sync_copy(dst_hbm.at[pl.ds(wid * CPW, CPW)], didx)

        @pl.when(sid == 0)
        def _():
            t0 = NSUB * RPS
            pltpu.sync_copy(z_hbm.at[pl.ds(t0, TAIL)], acc_sp.at[pl.ds(t0, TAIL)])
        plsc.subcore_barrier()

        def step(jj, carry):
            j0 = jj * 2
            j1 = j0 + 1

            @pl.when(jj > 0)
            def _():
                pltpu.make_async_copy(
                    ones_v, acc_sp.at[didx.at[j0 - 2]], sem_s0).wait()
                pltpu.make_async_copy(
                    ones_v, acc_sp.at[didx.at[j1 - 2]], sem_s1).wait()
            pltpu.async_copy(ones_v, acc_sp.at[didx.at[j0]], sem_s0, add=True)
            pltpu.async_copy(ones_v, acc_sp.at[didx.at[j1]], sem_s1, add=True)
            return carry

        lax.fori_loop(0, CPW // 2, step, 0)
        pltpu.make_async_copy(ones_v, acc_sp.at[didx.at[CPW - 2]], sem_s0).wait()
        pltpu.make_async_copy(ones_v, acc_sp.at[didx.at[CPW - 1]], sem_s1).wait()
        plsc.subcore_barrier()
        pltpu.sync_copy(acc_sp.at[pl.ds(base, RPS)],
                        out_hbm.at[cid, pl.ds(base, RPS)])

        @pl.when(sid == 0)
        def _():
            t0 = NSUB * RPS
            pltpu.sync_copy(acc_sp.at[pl.ds(t0, TAIL)],
                            out_hbm.at[cid, pl.ds(t0, TAIL)])

    return degk


def _agg_build():
    @functools.partial(
        pl.kernel,
        mesh=plsc.VectorSubcoreMesh(core_axis_name="c", subcore_axis_name="s"),
        out_type=jax.ShapeDtypeStruct((NCORE, N, D), jnp.float32),
        scratch_types=[
            pltpu.VMEM_SHARED((NPAD, D), jnp.float32),
            pltpu.VMEM((2, BLK, CHUNK), jnp.int32),
            pltpu.VMEM((2, BLK, CHUNK), jnp.int32),
            pltpu.VMEM((CHUNK, D), jnp.float32),
            pltpu.VMEM((CHUNK, D), jnp.float32),
            pltpu.SemaphoreType.DMA,
            pltpu.SemaphoreType.DMA,
            pltpu.SemaphoreType.DMA,
            pltpu.SemaphoreType.DMA,
        ],
    )
    def agg(h_hbm, src_hbm, dst_hbm, z_hbm,
            out_hbm,
            acc_sp, sidx, didx, rows0, rows1,
            sem_g0, sem_g1, sem_s0, sem_s1):
        cid = lax.axis_index("c")
        sid = lax.axis_index("s")
        wid = cid * NSUB + sid
        base = sid * RPS
        pltpu.sync_copy(src_hbm.at[pl.ds(wid * CPW, BLK)], sidx.at[0])
        pltpu.sync_copy(dst_hbm.at[pl.ds(wid * CPW, BLK)], didx.at[0])
        pltpu.async_copy(h_hbm.at[sidx.at[0, 0]], rows0, sem_g0)
        pltpu.sync_copy(z_hbm.at[pl.ds(base, RPS)], acc_sp.at[pl.ds(base, RPS)])

        @pl.when(sid == 0)
        def _():
            t0 = NSUB * RPS
            pltpu.sync_copy(z_hbm.at[pl.ds(t0, TAIL)], acc_sp.at[pl.ds(t0, TAIL)])
        plsc.subcore_barrier()

        rows = (rows0, rows1)
        sem_g = (sem_g0, sem_g1)
        sem_s = (sem_s0, sem_s1)

        for blk in range(NBLK):
            bb = blk % 2
            for k in range(BLK):
                p = k % 2
                if k == 0:
                    if blk > 0:
                        pltpu.make_async_copy(
                            rows[1 - p],
                            acc_sp.at[didx.at[1 - bb, BLK - 1]],
                            sem_s[1 - p]).wait()
                    if blk < NBLK - 1:
                        off = wid * CPW + (blk + 1) * BLK
                        pltpu.sync_copy(src_hbm.at[pl.ds(off, BLK)],
                                        sidx.at[1 - bb])
                        pltpu.sync_copy(dst_hbm.at[pl.ds(off, BLK)],
                                        didx.at[1 - bb])
                else:
                    pltpu.make_async_copy(
                        rows[1 - p], acc_sp.at[didx.at[bb, k - 1]],
                        sem_s[1 - p]).wait()
                if k == BLK - 1:
                    if blk < NBLK - 1:
                        pltpu.async_copy(h_hbm.at[sidx.at[1 - bb, 0]],
                                         rows[1 - p], sem_g[1 - p])
                else:
                    pltpu.async_copy(h_hbm.at[sidx.at[bb, k + 1]],
                                     rows[1 - p], sem_g[1 - p])
                pltpu.make_async_copy(h_hbm.at[sidx.at[bb, k]],
                                      rows[p], sem_g[p]).wait()
                pltpu.async_copy(rows[p], acc_sp.at[didx.at[bb, k]],
                                 sem_s[p], add=True)

        pltpu.make_async_copy(rows1,
                              acc_sp.at[didx.at[(NBLK - 1) % 2, BLK - 1]],
                              sem_s1).wait()
        plsc.subcore_barrier()
        pltpu.sync_copy(acc_sp.at[pl.ds(base, RPS)],
                        out_hbm.at[cid, pl.ds(base, RPS)])

        @pl.when(sid == 0)
        def _():
            t0 = NSUB * RPS
            pltpu.sync_copy(acc_sp.at[pl.ds(t0, TAIL)],
                            out_hbm.at[cid, pl.ds(t0, TAIL)])

    return agg


_sc_cache = {}


def _deg(*args):
    if "deg" not in _sc_cache:
        _sc_cache["deg"] = _deg_build()
    return _sc_cache["deg"](*args)


def _agg(*args):
    if "agg" not in _sc_cache:
        _sc_cache["agg"] = _agg_build()
    return _sc_cache["agg"](*args)


def _deginv_body(dacc_ref, out_ref):
    d = dacc_ref[0, :, 0:1] + dacc_ref[1, :, 0:1]
    out_ref[...] = jnp.broadcast_to(1.0 / jnp.clip(d, 1.0, None), (RB, D))


def _deginv(dacc):
    return pl.pallas_call(
        _deginv_body,
        grid=(GRID,),
        in_specs=[pl.BlockSpec((NCORE, RB, D), lambda i: (0, i, 0))],
        out_specs=pl.BlockSpec((RB, D), lambda i: (i, 0)),
        out_shape=jax.ShapeDtypeStruct((N, D), jnp.float32),
    )(dacc)


def _hr_body(h_ref, wr_ref, b_ref, out_ref):
    out_ref[...] = jnp.dot(h_ref[...], wr_ref[...],
                           preferred_element_type=jnp.float32) + b_ref[...]


def _hr(h, wr, b):
    return pl.pallas_call(
        _hr_body,
        grid=(GRID,),
        in_specs=[
            pl.BlockSpec((RB, D), lambda i: (i, 0)),
            pl.BlockSpec((D, D), lambda i: (0, 0)),
            pl.BlockSpec((1, D), lambda i: (0, 0)),
        ],
        out_specs=pl.BlockSpec((RB, D), lambda i: (i, 0)),
        out_shape=jax.ShapeDtypeStruct((N, D), jnp.float32),
    )(h, wr, b)


def _layer_body(acc_ref, dinv_ref, hr_ref, wl_ref, g_ref,
                beta_ref, out_ref, pre_scr, s1_scr, s2_scr):
    ph = pl.program_id(0)
    i = pl.program_id(1)

    @pl.when(ph == 0)
    def _():
        mean = (acc_ref[0] + acc_ref[1]) * dinv_ref[...]
        pre = jnp.dot(mean, wl_ref[...], preferred_element_type=jnp.float32)
        pre = pre + hr_ref[...]
        pre_scr[pl.ds(i * RB, RB), :] = pre

        @pl.when(i == 0)
        def _():
            s1_scr[...] = jnp.zeros((1, D), jnp.float32)
            s2_scr[...] = jnp.zeros((1, D), jnp.float32)
        s1_scr[...] += jnp.sum(pre, axis=0, keepdims=True)
        s2_scr[...] += jnp.sum(pre * pre, axis=0, keepdims=True)

    @pl.when(ph == 1)
    def _():
        mu = s1_scr[...] / N
        var = s2_scr[...] / N - mu * mu
        scale = g_ref[...] * lax.rsqrt(var + EPS)
        pre = pre_scr[pl.ds(i * RB, RB), :]
        out_ref[...] = jnp.maximum((pre - mu) * scale + beta_ref[...], 0.0)


def _layer(acc, dinv, hr, wl, g, beta):
    blk = lambda ph, i: (i * (1 - ph), 0)
    return pl.pallas_call(
        _layer_body,
        grid=(2, GRID),
        in_specs=[
            pl.BlockSpec((NCORE, RB, D), lambda ph, i: (0, i * (1 - ph), 0)),
            pl.BlockSpec((RB, D), blk),
            pl.BlockSpec((RB, D), blk),
            pl.BlockSpec((D, D), lambda ph, i: (0, 0)),
            pl.BlockSpec((1, D), lambda ph, i: (0, 0)),
            pl.BlockSpec((1, D), lambda ph, i: (0, 0)),
        ],
        out_specs=pl.BlockSpec((RB, D), lambda ph, i: (i, 0)),
        out_shape=jax.ShapeDtypeStruct((N, D), jnp.float32),
        scratch_shapes=[
            pltpu.VMEM((N, D), jnp.float32),
            pltpu.VMEM((1, D), jnp.float32),
            pltpu.VMEM((1, D), jnp.float32),
        ],
    )(acc, dinv, hr, wl, g, beta)


def _out_body(acc_ref, dinv_ref, hr_ref, wl_ref, out_ref):
    mean = (acc_ref[0] + acc_ref[1]) * dinv_ref[...]
    pre = jnp.dot(mean, wl_ref[...], preferred_element_type=jnp.float32)
    pre = pre + hr_ref[...]
    m = jnp.max(pre, axis=1, keepdims=True)
    e = jnp.exp(pre - m)
    s = jnp.sum(e, axis=1, keepdims=True)
    out_ref[...] = pre - m - jnp.log(s)


def _out_layer(acc, dinv, hr, wl):
    return pl.pallas_call(
        _out_body,
        grid=(GRID,),
        in_specs=[
            pl.BlockSpec((NCORE, RB, D), lambda i: (0, i, 0)),
            pl.BlockSpec((RB, D), lambda i: (i, 0)),
            pl.BlockSpec((RB, D), lambda i: (i, 0)),
            pl.BlockSpec((D, D), lambda i: (0, 0)),
        ],
        out_specs=pl.BlockSpec((RB, D), lambda i: (i, 0)),
        out_shape=jax.ShapeDtypeStruct((N, D), jnp.float32),
    )(acc, dinv, hr, wl)


def kernel(x, edge_index, Wl1, Wr1, b1, g1, beta1, Wl2, Wr2, b2, g2, beta2,
           Wl3, Wr3, b3, g3, beta3, Wl4, Wr4, b4):
    pad = EPAD - E
    tr = jnp.arange(pad, dtype=jnp.int32) % NTRASH
    src = jnp.concatenate([edge_index[0], tr]).reshape(NCHUNK, CHUNK)
    dst = jnp.concatenate([edge_index[1], N + tr]).reshape(NCHUNK, CHUNK)
    zeros = jnp.zeros((N, D), jnp.float32)
    ones = jnp.ones((CHUNK, D), jnp.float32)
    r = lambda v: v.reshape(1, D)

    dacc = _deg(dst, zeros, ones)
    acc1 = _agg(x, src, dst, zeros)
    hr1 = _hr(x, Wr1, r(b1))
    dinv = _deginv(dacc)
    h1 = _layer(acc1, dinv, hr1, Wl1, r(g1), r(beta1))

    acc2 = _agg(h1, src, dst, zeros)
    hr2 = _hr(h1, Wr2, r(b2))
    h2 = _layer(acc2, dinv, hr2, Wl2, r(g2), r(beta2))

    acc3 = _agg(h2, src, dst, zeros)
    hr3 = _hr(h2, Wr3, r(b3))
    h3 = _layer(acc3, dinv, hr3, Wl3, r(g3), r(beta3))

    acc4 = _agg(h3, src, dst, zeros)
    hr4 = _hr(h3, Wr4, r(b4))
    return _out_layer(acc4, dinv, hr4, Wl4)


# speedup vs baseline: 1.0282x; 1.0027x over previous
import functools

import jax
import jax.numpy as jnp
from jax import lax
from jax.experimental import pallas as pl
from jax.experimental.pallas import tpu as pltpu
from jax.experimental.pallas import tpu_sc as plsc

N = 10000
E = 320000
D = 128
NCORE = 2
NSUB = 16
NW = NCORE * NSUB
CHUNK = 128
CPW = 80
BLK = 16
NBLK = CPW // BLK
NCHUNK = NW * CPW
EPAD = NCHUNK * CHUNK
NTRASH = 64
NPAD = N + NTRASH
RPS = 624
TAIL = N - NSUB * RPS
RB = 1000
GRID = N // RB
EPS = 1e-5

def _deg_build():
    @functools.partial(
        pl.kernel,
        mesh=plsc.VectorSubcoreMesh(core_axis_name="c", subcore_axis_name="s"),
        out_type=jax.ShapeDtypeStruct((NCORE, N, D), jnp.float32),
        scratch_types=[
            pltpu.VMEM_SHARED((NPAD, D), jnp.float32),
            pltpu.VMEM((CPW, CHUNK), jnp.int32),
            pltpu.VMEM((CHUNK, D), jnp.float32),
            pltpu.SemaphoreType.DMA,
            pltpu.SemaphoreType.DMA,
        ],
    )
    def degk(dst_hbm, z_hbm, ones_hbm, out_hbm, acc_sp, didx, ones_v,
             sem_s0, sem_s1):
        cid = lax.axis_index("c")
        sid = lax.axis_index("s")
        wid = cid * NSUB + sid
        base = sid * RPS
        pltpu.sync_copy(z_hbm.at[pl.ds(base, RPS)], acc_sp.at[pl.ds(base, RPS)])
        pltpu.sync_copy(ones_hbm, ones_v)
        pltpu.sync_copy(dst_hbm.at[pl.ds(wid * CPW, CPW)], didx)

        @pl.when(sid == 0)
        def _():
            t0 = NSUB * RPS
            pltpu.sync_copy(z_hbm.at[pl.ds(t0, TAIL)], acc_sp.at[pl.ds(t0, TAIL)])
        plsc.subcore_barrier()

        def step(jj, carry):
            j0 = jj * 2
            j1 = j0 + 1

            @pl.when(jj > 0)
            def _():
                pltpu.make_async_copy(
                    ones_v, acc_sp.at[didx.at[j0 - 2]], sem_s0).wait()
                pltpu.make_async_copy(
                    ones_v, acc_sp.at[didx.at[j1 - 2]], sem_s1).wait()
            pltpu.async_copy(ones_v, acc_sp.at[didx.at[j0]], sem_s0, add=True)
            pltpu.async_copy(ones_v, acc_sp.at[didx.at[j1]], sem_s1, add=True)
            return carry

        lax.fori_loop(0, CPW // 2, step, 0)
        pltpu.make_async_copy(ones_v, acc_sp.at[didx.at[CPW - 2]], sem_s0).wait()
        pltpu.make_async_copy(ones_v, acc_sp.at[didx.at[CPW - 1]], sem_s1).wait()
        plsc.subcore_barrier()
        pltpu.sync_copy(acc_sp.at[pl.ds(base, RPS)],
                        out_hbm.at[cid, pl.ds(base, RPS)])

        @pl.when(sid == 0)
        def _():
            t0 = NSUB * RPS
            pltpu.sync_copy(acc_sp.at[pl.ds(t0, TAIL)],
                            out_hbm.at[cid, pl.ds(t0, TAIL)])

    return degk


def _agg_build():
    @functools.partial(
        pl.kernel,
        mesh=plsc.VectorSubcoreMesh(core_axis_name="c", subcore_axis_name="s"),
        out_type=jax.ShapeDtypeStruct((NCORE, N, D), jnp.float32),
        scratch_types=[
            pltpu.VMEM_SHARED((NPAD, D), jnp.float32),
            pltpu.VMEM((2, BLK, CHUNK), jnp.int32),
            pltpu.VMEM((2, BLK, CHUNK), jnp.int32),
            pltpu.VMEM((CHUNK, D), jnp.float32),
            pltpu.VMEM((CHUNK, D), jnp.float32),
            pltpu.SemaphoreType.DMA,
            pltpu.SemaphoreType.DMA,
            pltpu.SemaphoreType.DMA,
            pltpu.SemaphoreType.DMA,
        ],
    )
    def agg(h_hbm, src_hbm, dst_hbm, z_hbm,
            out_hbm,
            acc_sp, sidx, didx, rows0, rows1,
            sem_g0, sem_g1, sem_s0, sem_s1):
        cid = lax.axis_index("c")
        sid = lax.axis_index("s")
        wid = cid * NSUB + sid
        base = sid * RPS
        pltpu.sync_copy(src_hbm.at[pl.ds(wid * CPW, BLK)], sidx.at[0])
        pltpu.sync_copy(dst_hbm.at[pl.ds(wid * CPW, BLK)], didx.at[0])
        pltpu.async_copy(h_hbm.at[sidx.at[0, 0]], rows0, sem_g0)
        pltpu.async_copy(h_hbm.at[sidx.at[0, 1]], rows1, sem_g1)
        pltpu.sync_copy(z_hbm.at[pl.ds(base, RPS)], acc_sp.at[pl.ds(base, RPS)])

        @pl.when(sid == 0)
        def _():
            t0 = NSUB * RPS
            pltpu.sync_copy(z_hbm.at[pl.ds(t0, TAIL)], acc_sp.at[pl.ds(t0, TAIL)])
        plsc.subcore_barrier()

        rows = (rows0, rows1)
        sem_g = (sem_g0, sem_g1)
        sem_s = (sem_s0, sem_s1)

        for blk in range(NBLK):
            bb = blk % 2
            for k in range(BLK):
                p = k % 2
                if k == 0:
                    if blk > 0:
                        pltpu.make_async_copy(
                            rows[1 - p],
                            acc_sp.at[didx.at[1 - bb, BLK - 1]],
                            sem_s[1 - p]).wait()
                    if blk < NBLK - 1:
                        off = wid * CPW + (blk + 1) * BLK
                        pltpu.sync_copy(src_hbm.at[pl.ds(off, BLK)],
                                        sidx.at[1 - bb])
                        pltpu.sync_copy(dst_hbm.at[pl.ds(off, BLK)],
                                        didx.at[1 - bb])
                else:
                    pltpu.make_async_copy(
                        rows[1 - p], acc_sp.at[didx.at[bb, k - 1]],
                        sem_s[1 - p]).wait()
                if k == BLK - 1:
                    if blk < NBLK - 1:
                        pltpu.async_copy(h_hbm.at[sidx.at[1 - bb, 0]],
                                         rows[1 - p], sem_g[1 - p])
                elif not (blk == 0 and k == 0):
                    pltpu.async_copy(h_hbm.at[sidx.at[bb, k + 1]],
                                     rows[1 - p], sem_g[1 - p])
                pltpu.make_async_copy(h_hbm.at[sidx.at[bb, k]],
                                      rows[p], sem_g[p]).wait()
                pltpu.async_copy(rows[p], acc_sp.at[didx.at[bb, k]],
                                 sem_s[p], add=True)

        pltpu.make_async_copy(rows1,
                              acc_sp.at[didx.at[(NBLK - 1) % 2, BLK - 1]],
                              sem_s1).wait()
        plsc.subcore_barrier()
        pltpu.sync_copy(acc_sp.at[pl.ds(base, RPS)],
                        out_hbm.at[cid, pl.ds(base, RPS)])

        @pl.when(sid == 0)
        def _():
            t0 = NSUB * RPS
            pltpu.sync_copy(acc_sp.at[pl.ds(t0, TAIL)],
                            out_hbm.at[cid, pl.ds(t0, TAIL)])

    return agg


_sc_cache = {}


def _deg(*args):
    if "deg" not in _sc_cache:
        _sc_cache["deg"] = _deg_build()
    return _sc_cache["deg"](*args)


def _agg(*args):
    if "agg" not in _sc_cache:
        _sc_cache["agg"] = _agg_build()
    return _sc_cache["agg"](*args)


def _deginv_body(dacc_ref, out_ref):
    d = dacc_ref[0, :, 0:1] + dacc_ref[1, :, 0:1]
    out_ref[...] = jnp.broadcast_to(1.0 / jnp.clip(d, 1.0, None), (RB, D))


def _deginv(dacc):
    return pl.pallas_call(
        _deginv_body,
        grid=(GRID,),
        in_specs=[pl.BlockSpec((NCORE, RB, D), lambda i: (0, i, 0))],
        out_specs=pl.BlockSpec((RB, D), lambda i: (i, 0)),
        out_shape=jax.ShapeDtypeStruct((N, D), jnp.float32),
    )(dacc)


def _layer_body(acc_ref, dinv_ref, h_ref, wl_ref, wr_ref, b_ref, g_ref,
                beta_ref, out_ref, pre_scr, s1_scr, s2_scr):
    ph = pl.program_id(0)
    i = pl.program_id(1)

    @pl.when(ph == 0)
    def _():
        mean = (acc_ref[0] + acc_ref[1]) * dinv_ref[...]
        pre = jnp.dot(mean, wl_ref[...], preferred_element_type=jnp.float32)
        pre = pre + jnp.dot(h_ref[...], wr_ref[...],
                            preferred_element_type=jnp.float32)
        pre = pre + b_ref[...]
        pre_scr[pl.ds(i * RB, RB), :] = pre

        @pl.when(i == 0)
        def _():
            s1_scr[...] = jnp.zeros((1, D), jnp.float32)
            s2_scr[...] = jnp.zeros((1, D), jnp.float32)
        s1_scr[...] += jnp.sum(pre, axis=0, keepdims=True)
        s2_scr[...] += jnp.sum(pre * pre, axis=0, keepdims=True)

    @pl.when(ph == 1)
    def _():
        mu = s1_scr[...] / N
        var = s2_scr[...] / N - mu * mu
        scale = g_ref[...] * lax.rsqrt(var + EPS)
        pre = pre_scr[pl.ds(i * RB, RB), :]
        out_ref[...] = jnp.maximum((pre - mu) * scale + beta_ref[...], 0.0)


def _layer(acc, dinv, h, wl, wr, b, g, beta):
    blk = lambda ph, i: (i * (1 - ph), 0)
    return pl.pallas_call(
        _layer_body,
        grid=(2, GRID),
        in_specs=[
            pl.BlockSpec((NCORE, RB, D), lambda ph, i: (0, i * (1 - ph), 0)),
            pl.BlockSpec((RB, D), blk),
            pl.BlockSpec((RB, D), blk),
            pl.BlockSpec((D, D), lambda ph, i: (0, 0)),
            pl.BlockSpec((D, D), lambda ph, i: (0, 0)),
            pl.BlockSpec((1, D), lambda ph, i: (0, 0)),
            pl.BlockSpec((1, D), lambda ph, i: (0, 0)),
            pl.BlockSpec((1, D), lambda ph, i: (0, 0)),
        ],
        out_specs=pl.BlockSpec((RB, D), lambda ph, i: (i, 0)),
        out_shape=jax.ShapeDtypeStruct((N, D), jnp.float32),
        scratch_shapes=[
            pltpu.VMEM((N, D), jnp.float32),
            pltpu.VMEM((1, D), jnp.float32),
            pltpu.VMEM((1, D), jnp.float32),
        ],
    )(acc, dinv, h, wl, wr, b, g, beta)


def _out_body(acc_ref, dinv_ref, h_ref, wl_ref, wr_ref, b_ref, out_ref):
    mean = (acc_ref[0] + acc_ref[1]) * dinv_ref[...]
    pre = jnp.dot(mean, wl_ref[...], preferred_element_type=jnp.float32)
    pre = pre + jnp.dot(h_ref[...], wr_ref[...],
                        preferred_element_type=jnp.float32)
    pre = pre + b_ref[...]
    m = jnp.max(pre, axis=1, keepdims=True)
    e = jnp.exp(pre - m)
    s = jnp.sum(e, axis=1, keepdims=True)
    out_ref[...] = pre - m - jnp.log(s)


def _out_layer(acc, dinv, h, wl, wr, b):
    return pl.pallas_call(
        _out_body,
        grid=(GRID,),
        in_specs=[
            pl.BlockSpec((NCORE, RB, D), lambda i: (0, i, 0)),
            pl.BlockSpec((RB, D), lambda i: (i, 0)),
            pl.BlockSpec((RB, D), lambda i: (i, 0)),
            pl.BlockSpec((D, D), lambda i: (0, 0)),
            pl.BlockSpec((D, D), lambda i: (0, 0)),
            pl.BlockSpec((1, D), lambda i: (0, 0)),
        ],
        out_specs=pl.BlockSpec((RB, D), lambda i: (i, 0)),
        out_shape=jax.ShapeDtypeStruct((N, D), jnp.float32),
    )(acc, dinv, h, wl, wr, b)


def kernel(x, edge_index, Wl1, Wr1, b1, g1, beta1, Wl2, Wr2, b2, g2, beta2,
           Wl3, Wr3, b3, g3, beta3, Wl4, Wr4, b4):
    pad = EPAD - E
    tr = jnp.arange(pad, dtype=jnp.int32) % NTRASH
    src = jnp.concatenate([edge_index[0], tr]).reshape(NCHUNK, CHUNK)
    dst = jnp.concatenate([edge_index[1], N + tr]).reshape(NCHUNK, CHUNK)
    zeros = jnp.zeros((N, D), jnp.float32)
    ones = jnp.ones((CHUNK, D), jnp.float32)
    r = lambda v: v.reshape(1, D)

    dacc = _deg(dst, zeros, ones)
    acc1 = _agg(x, src, dst, zeros)
    dinv = _deginv(dacc)
    h1 = _layer(acc1, dinv, x, Wl1, Wr1, r(b1), r(g1), r(beta1))

    acc2 = _agg(h1, src, dst, zeros)
    h2 = _layer(acc2, dinv, h1, Wl2, Wr2, r(b2), r(g2), r(beta2))

    acc3 = _agg(h2, src, dst, zeros)
    h3 = _layer(acc3, dinv, h2, Wl3, Wr3, r(b3), r(g3), r(beta3))

    acc4 = _agg(h3, src, dst, zeros)
    return _out_layer(acc4, dinv, h3, Wl4, Wr4, r(b4))


# speedup vs baseline: 1.0304x; 1.0022x over previous
import functools

import jax
import jax.numpy as jnp
from jax import lax
from jax.experimental import pallas as pl
from jax.experimental.pallas import tpu as pltpu
from jax.experimental.pallas import tpu_sc as plsc

N = 10000
E = 320000
D = 128
NCORE = 2
NSUB = 16
NW = NCORE * NSUB
CHUNK = 128
CPW = 80
BLK = 16
NBLK = CPW // BLK
NCHUNK = NW * CPW
EPAD = NCHUNK * CHUNK
NTRASH = 64
NPAD = N + NTRASH
RPS = 624
TAIL = N - NSUB * RPS
RB = 1000
GRID = N // RB
EPS = 1e-5

def _deg_build():
    @functools.partial(
        pl.kernel,
        mesh=plsc.VectorSubcoreMesh(core_axis_name="c", subcore_axis_name="s"),
        out_type=jax.ShapeDtypeStruct((NCORE, N, D), jnp.float32),
        scratch_types=[
            pltpu.VMEM_SHARED((NPAD, D), jnp.float32),
            pltpu.VMEM((CPW, CHUNK), jnp.int32),
            pltpu.VMEM((CHUNK, D), jnp.float32),
            pltpu.SemaphoreType.DMA,
            pltpu.SemaphoreType.DMA,
            pltpu.SemaphoreType.DMA,
            pltpu.SemaphoreType.DMA,
        ],
    )
    def degk(dst_hbm, z_hbm, ones_hbm, out_hbm, acc_sp, didx, ones_v,
             sem_s0, sem_s1, sem_s2, sem_s3):
        cid = lax.axis_index("c")
        sid = lax.axis_index("s")
        wid = cid * NSUB + sid
        base = sid * RPS
        pltpu.sync_copy(z_hbm.at[pl.ds(base, RPS)], acc_sp.at[pl.ds(base, RPS)])
        pltpu.sync_copy(ones_hbm, ones_v)
        pltpu.sync_copy(dst_hbm.at[pl.ds(wid * CPW, CPW)], didx)

        @pl.when(sid == 0)
        def _():
            t0 = NSUB * RPS
            pltpu.sync_copy(z_hbm.at[pl.ds(t0, TAIL)], acc_sp.at[pl.ds(t0, TAIL)])
        plsc.subcore_barrier()

        sems = (sem_s0, sem_s1, sem_s2, sem_s3)

        def step(jj, carry):
            for q in range(4):
                j = jj * 4 + q

                @pl.when(jj > 0)
                def _():
                    pltpu.make_async_copy(
                        ones_v, acc_sp.at[didx.at[j - 4]], sems[q]).wait()
                pltpu.async_copy(ones_v, acc_sp.at[didx.at[j]], sems[q],
                                 add=True)
            return carry

        lax.fori_loop(0, CPW // 4, step, 0)
        for q in range(4):
            pltpu.make_async_copy(
                ones_v, acc_sp.at[didx.at[CPW - 4 + q]], sems[q]).wait()
        plsc.subcore_barrier()
        pltpu.sync_copy(acc_sp.at[pl.ds(base, RPS)],
                        out_hbm.at[cid, pl.ds(base, RPS)])

        @pl.when(sid == 0)
        def _():
            t0 = NSUB * RPS
            pltpu.sync_copy(acc_sp.at[pl.ds(t0, TAIL)],
                            out_hbm.at[cid, pl.ds(t0, TAIL)])

    return degk


def _agg_build():
    @functools.partial(
        pl.kernel,
        mesh=plsc.VectorSubcoreMesh(core_axis_name="c", subcore_axis_name="s"),
        out_type=jax.ShapeDtypeStruct((NCORE, N, D), jnp.float32),
        scratch_types=[
            pltpu.VMEM_SHARED((NPAD, D), jnp.float32),
            pltpu.VMEM((2, BLK, CHUNK), jnp.int32),
            pltpu.VMEM((2, BLK, CHUNK), jnp.int32),
            pltpu.VMEM((CHUNK, D), jnp.float32),
            pltpu.VMEM((CHUNK, D), jnp.float32),
            pltpu.SemaphoreType.DMA,
            pltpu.SemaphoreType.DMA,
            pltpu.SemaphoreType.DMA,
            pltpu.SemaphoreType.DMA,
        ],
    )
    def agg(h_hbm, src_hbm, dst_hbm, z_hbm,
            out_hbm,
            acc_sp, sidx, didx, rows0, rows1,
            sem_g0, sem_g1, sem_s0, sem_s1):
        cid = lax.axis_index("c")
        sid = lax.axis_index("s")
        wid = cid * NSUB + sid
        base = sid * RPS
        pltpu.sync_copy(src_hbm.at[pl.ds(wid * CPW, BLK)], sidx.at[0])
        pltpu.sync_copy(dst_hbm.at[pl.ds(wid * CPW, BLK)], didx.at[0])
        pltpu.async_copy(h_hbm.at[sidx.at[0, 0]], rows0, sem_g0)
        pltpu.async_copy(h_hbm.at[sidx.at[0, 1]], rows1, sem_g1)
        pltpu.sync_copy(z_hbm.at[pl.ds(base, RPS)], acc_sp.at[pl.ds(base, RPS)])

        @pl.when(sid == 0)
        def _():
            t0 = NSUB * RPS
            pltpu.sync_copy(z_hbm.at[pl.ds(t0, TAIL)], acc_sp.at[pl.ds(t0, TAIL)])
        plsc.subcore_barrier()

        rows = (rows0, rows1)
        sem_g = (sem_g0, sem_g1)
        sem_s = (sem_s0, sem_s1)

        for blk in range(NBLK):
            bb = blk % 2
            for k in range(BLK):
                p = k % 2
                if k == 0:
                    if blk > 0:
                        pltpu.make_async_copy(
                            rows[1 - p],
                            acc_sp.at[didx.at[1 - bb, BLK - 1]],
                            sem_s[1 - p]).wait()
                    if blk < NBLK - 1:
                        off = wid * CPW + (blk + 1) * BLK
                        pltpu.sync_copy(src_hbm.at[pl.ds(off, BLK)],
                                        sidx.at[1 - bb])
                        pltpu.sync_copy(dst_hbm.at[pl.ds(off, BLK)],
                                        didx.at[1 - bb])
                else:
                    pltpu.make_async_copy(
                        rows[1 - p], acc_sp.at[didx.at[bb, k - 1]],
                        sem_s[1 - p]).wait()
                if k == BLK - 1:
                    if blk < NBLK - 1:
                        pltpu.async_copy(h_hbm.at[sidx.at[1 - bb, 0]],
                                         rows[1 - p], sem_g[1 - p])
                elif not (blk == 0 and k == 0):
                    pltpu.async_copy(h_hbm.at[sidx.at[bb, k + 1]],
                                     rows[1 - p], sem_g[1 - p])
                pltpu.make_async_copy(h_hbm.at[sidx.at[bb, k]],
                                      rows[p], sem_g[p]).wait()
                pltpu.async_copy(rows[p], acc_sp.at[didx.at[bb, k]],
                                 sem_s[p], add=True)

        pltpu.make_async_copy(rows1,
                              acc_sp.at[didx.at[(NBLK - 1) % 2, BLK - 1]],
                              sem_s1).wait()
        plsc.subcore_barrier()
        pltpu.sync_copy(acc_sp.at[pl.ds(base, RPS)],
                        out_hbm.at[cid, pl.ds(base, RPS)])

        @pl.when(sid == 0)
        def _():
            t0 = NSUB * RPS
            pltpu.sync_copy(acc_sp.at[pl.ds(t0, TAIL)],
                            out_hbm.at[cid, pl.ds(t0, TAIL)])

    return agg


_sc_cache = {}


def _deg(*args):
    if "deg" not in _sc_cache:
        _sc_cache["deg"] = _deg_build()
    return _sc_cache["deg"](*args)


def _agg(*args):
    if "agg" not in _sc_cache:
        _sc_cache["agg"] = _agg_build()
    return _sc_cache["agg"](*args)


def _deginv_body(dacc_ref, out_ref):
    d = dacc_ref[0, :, 0:1] + dacc_ref[1, :, 0:1]
    out_ref[...] = jnp.broadcast_to(1.0 / jnp.clip(d, 1.0, None), (RB, D))


def _deginv(dacc):
    return pl.pallas_call(
        _deginv_body,
        grid=(GRID,),
        in_specs=[pl.BlockSpec((NCORE, RB, D), lambda i: (0, i, 0))],
        out_specs=pl.BlockSpec((RB, D), lambda i: (i, 0)),
        out_shape=jax.ShapeDtypeStruct((N, D), jnp.float32),
    )(dacc)


def _layer_body(acc_ref, dinv_ref, h_ref, wl_ref, wr_ref, b_ref, g_ref,
                beta_ref, out_ref, pre_scr, s1_scr, s2_scr):
    ph = pl.program_id(0)
    i = pl.program_id(1)

    @pl.when(ph == 0)
    def _():
        mean = (acc_ref[0] + acc_ref[1]) * dinv_ref[...]
        pre = jnp.dot(mean, wl_ref[...], preferred_element_type=jnp.float32)
        pre = pre + jnp.dot(h_ref[...], wr_ref[...],
                            preferred_element_type=jnp.float32)
        pre = pre + b_ref[...]
        pre_scr[pl.ds(i * RB, RB), :] = pre

        @pl.when(i == 0)
        def _():
            s1_scr[...] = jnp.zeros((1, D), jnp.float32)
            s2_scr[...] = jnp.zeros((1, D), jnp.float32)
        s1_scr[...] += jnp.sum(pre, axis=0, keepdims=True)
        s2_scr[...] += jnp.sum(pre * pre, axis=0, keepdims=True)

    @pl.when(ph == 1)
    def _():
        mu = s1_scr[...] / N
        var = s2_scr[...] / N - mu * mu
        scale = g_ref[...] * lax.rsqrt(var + EPS)
        pre = pre_scr[pl.ds(i * RB, RB), :]
        out_ref[...] = jnp.maximum((pre - mu) * scale + beta_ref[...], 0.0)


def _layer(acc, dinv, h, wl, wr, b, g, beta):
    blk = lambda ph, i: (i * (1 - ph), 0)
    return pl.pallas_call(
        _layer_body,
        grid=(2, GRID),
        in_specs=[
            pl.BlockSpec((NCORE, RB, D), lambda ph, i: (0, i * (1 - ph), 0)),
            pl.BlockSpec((RB, D), blk),
            pl.BlockSpec((RB, D), blk),
            pl.BlockSpec((D, D), lambda ph, i: (0, 0)),
            pl.BlockSpec((D, D), lambda ph, i: (0, 0)),
            pl.BlockSpec((1, D), lambda ph, i: (0, 0)),
            pl.BlockSpec((1, D), lambda ph, i: (0, 0)),
            pl.BlockSpec((1, D), lambda ph, i: (0, 0)),
        ],
        out_specs=pl.BlockSpec((RB, D), lambda ph, i: (i, 0)),
        out_shape=jax.ShapeDtypeStruct((N, D), jnp.float32),
        scratch_shapes=[
            pltpu.VMEM((N, D), jnp.float32),
            pltpu.VMEM((1, D), jnp.float32),
            pltpu.VMEM((1, D), jnp.float32),
        ],
    )(acc, dinv, h, wl, wr, b, g, beta)


def _out_body(acc_ref, dinv_ref, h_ref, wl_ref, wr_ref, b_ref, out_ref):
    mean = (acc_ref[0] + acc_ref[1]) * dinv_ref[...]
    pre = jnp.dot(mean, wl_ref[...], preferred_element_type=jnp.float32)
    pre = pre + jnp.dot(h_ref[...], wr_ref[...],
                        preferred_element_type=jnp.float32)
    pre = pre + b_ref[...]
    m = jnp.max(pre, axis=1, keepdims=True)
    e = jnp.exp(pre - m)
    s = jnp.sum(e, axis=1, keepdims=True)
    out_ref[...] = pre - m - jnp.log(s)


def _out_layer(acc, dinv, h, wl, wr, b):
    return pl.pallas_call(
        _out_body,
        grid=(GRID,),
        in_specs=[
            pl.BlockSpec((NCORE, RB, D), lambda i: (0, i, 0)),
            pl.BlockSpec((RB, D), lambda i: (i, 0)),
            pl.BlockSpec((RB, D), lambda i: (i, 0)),
            pl.BlockSpec((D, D), lambda i: (0, 0)),
            pl.BlockSpec((D, D), lambda i: (0, 0)),
            pl.BlockSpec((1, D), lambda i: (0, 0)),
        ],
        out_specs=pl.BlockSpec((RB, D), lambda i: (i, 0)),
        out_shape=jax.ShapeDtypeStruct((N, D), jnp.float32),
    )(acc, dinv, h, wl, wr, b)


def kernel(x, edge_index, Wl1, Wr1, b1, g1, beta1, Wl2, Wr2, b2, g2, beta2,
           Wl3, Wr3, b3, g3, beta3, Wl4, Wr4, b4):
    pad = EPAD - E
    tr = jnp.arange(pad, dtype=jnp.int32) % NTRASH
    src = jnp.concatenate([edge_index[0], tr]).reshape(NCHUNK, CHUNK)
    dst = jnp.concatenate([edge_index[1], N + tr]).reshape(NCHUNK, CHUNK)
    zeros = jnp.zeros((N, D), jnp.float32)
    ones = jnp.ones((CHUNK, D), jnp.float32)
    r = lambda v: v.reshape(1, D)

    dacc = _deg(dst, zeros, ones)
    acc1 = _agg(x, src, dst, zeros)
    dinv = _deginv(dacc)
    h1 = _layer(acc1, dinv, x, Wl1, Wr1, r(b1), r(g1), r(beta1))

    acc2 = _agg(h1, src, dst, zeros)
    h2 = _layer(acc2, dinv, h1, Wl2, Wr2, r(b2), r(g2), r(beta2))

    acc3 = _agg(h2, src, dst, zeros)
    h3 = _layer(acc3, dinv, h2, Wl3, Wr3, r(b3), r(g3), r(beta3))

    acc4 = _agg(h3, src, dst, zeros)
    return _out_layer(acc4, dinv, h3, Wl4, Wr4, r(b4))
